# baseline (speedup 1.0000x reference)
import functools

import numpy as np

import jax
import jax.numpy as jnp
from jax import lax
from jax.experimental import pallas as pl
from jax.experimental.pallas import tpu as pltpu
from jax.experimental.pallas import tpu_sc as plsc

B = 16384
L = 200
LP = 224
HALF = 112
D = 16
N_SUBJECTS = 100000
N_LANES = 16
NC, NS = 2, 16
NW = NC * NS
ROWS_PER_W = B // NW
RC = 8
N_CHUNKS = ROWS_PER_W // RC
NEG = -1e30

_SBLK = 2000


def _score_table_body(e_ref, w_ref, o_ref):
    o_ref[...] = jnp.sum(e_ref[...] * w_ref[...], axis=1, keepdims=True)


def _score_table(subj_emb, attn_w_row):
    grid = N_SUBJECTS // _SBLK
    return pl.pallas_call(
        _score_table_body,
        grid=(grid,),
        in_specs=[
            pl.BlockSpec((_SBLK, D), lambda i: (i, 0)),
            pl.BlockSpec((1, D), lambda i: (0, 0)),
        ],
        out_specs=pl.BlockSpec((_SBLK, 1), lambda i: (i, 0)),
        out_shape=jax.ShapeDtypeStruct((N_SUBJECTS, 1), jnp.float32),
    )(subj_emb, attn_w_row)


_GATHER_DNUMS = lax.GatherDimensionNumbers(
    offset_dims=(), collapsed_slice_dims=(0,), start_index_map=(0,))


def _splat_lane(vec, j):
    idx = jnp.full((N_LANES, 1), j, jnp.int32)
    return lax.gather(vec, idx, _GATHER_DNUMS, (1,),
                      mode=lax.GatherScatterMode.PROMISE_IN_BOUNDS)


def _bf16_halves(a32):
    pr = plsc.bitcast(a32, jnp.int32)
    lo = plsc.bitcast(jnp.left_shift(pr, 16), jnp.float32)
    hi = plsc.bitcast(pr & jnp.int32(-65536), jnp.float32)
    return lo, hi


def _vrecip(x):
    xi = plsc.bitcast(x, jnp.int32)
    magic = jnp.full((N_LANES,), 0x7EF311C3, jnp.int32)
    r = plsc.bitcast(magic - xi, jnp.float32)
    for _ in range(3):
        r = r * (2.0 - x * r)
    return r


def _pool_row(r, idx_ref, idxh_ref, rows_ref, s_v, sm_v):
    run_max = jnp.full((N_LANES,), NEG, jnp.float32)
    sms = []
    for h in range(2):
        for c in range(HALF // N_LANES):
            idx = idx_ref[r, h, pl.ds(c * N_LANES, N_LANES)]
            idxh = idxh_ref[r, h, pl.ds(c * N_LANES, N_LANES)]
            praw = plsc.load_gather(s_v, [idxh])
            sc = jnp.where(
                (idx & 1) == 0,
                plsc.bitcast(jnp.left_shift(praw, 16), jnp.float32),
                plsc.bitcast(praw & jnp.int32(-65536), jnp.float32))
            sm = jnp.where(idx != 0, sc, NEG)
            sms.append(sm)
            run_max = jnp.maximum(run_max, sm)
    m = jnp.max(run_max)
    zacc = jnp.zeros((N_LANES,), jnp.float32)
    accs = [jnp.zeros((2 * N_LANES,), jnp.bfloat16) for _ in range(8)]
    for k in range(LP // N_LANES):
        h, c = k // 7, k % 7
        idx = idx_ref[r, h, pl.ds(c * N_LANES, N_LANES)]
        sm = sms[k]
        e = jnp.exp(sm - m)
        e = jnp.where(sm != NEG, e, 0.0)
        zacc = zacc + e
        pv = idx & 1
        a = jnp.where(pv == 0, e, 0.0)
        b = e - a
        pp = (lax.shift_right_logical(plsc.bitcast(a, jnp.int32), 16)
              | (plsc.bitcast(b, jnp.int32) & jnp.int32(-65536)))
        for j in range(N_LANES):
            wb = plsc.bitcast(_splat_lane(pp, j), jnp.bfloat16)
            accs[j % 8] = (accs[j % 8]
                           + wb * rows_ref[h, c * N_LANES + j, :])
    z = jnp.maximum(jnp.sum(zacc), 1.0)
    acc = ((accs[0] + accs[1]) + (accs[2] + accs[3])) + ((accs[4] + accs[5]) + (accs[6] + accs[7]))
    return acc, z


def _sc_main(s_hbm, fav_hbm, book_hbm,
             uidx_hbm, iidx_hbm, emb_hbm,
             ubias_hbm, ibias_hbm, gb_hbm, out_hbm,
             s_v, fidx_v, bidx_v, fhidx_v, bhidx_v, urows_v, irows_v,
             uidx_v, iidx_v, ub_v, ib_v, out_v, gb_v, smu_v, smi_v,
             emb_sp, sem, sem2):
    wid = lax.axis_index("s") * NC + lax.axis_index("c")
    base = wid * ROWS_PER_W

    pltpu.sync_copy(s_hbm, s_v)
    pltpu.sync_copy(gb_hbm, gb_v)
    pltpu.sync_copy(uidx_hbm.at[pl.ds(wid * 4, 4)], uidx_v)
    pltpu.sync_copy(iidx_hbm.at[pl.ds(wid * 4, 4)], iidx_v)
    descs = []
    for c in range(4):
        descs.append(pltpu.async_copy(ubias_hbm.at[uidx_v.at[c]],
                                      ub_v.at[c], sem))
        descs.append(pltpu.async_copy(ibias_hbm.at[iidx_v.at[c]],
                                      ib_v.at[c], sem))
    for d in descs:
        d.wait()

    @pl.when(lax.axis_index("s") == 0)
    def _stage_table():
        pltpu.sync_copy(emb_hbm, emb_sp)

    plsc.subcore_barrier()

    def fire_row(r, par, psem):
        for h in range(2):
            pltpu.async_copy(emb_sp.at[fhidx_v.at[r, h]],
                             urows_v.at[par, h], psem)
            pltpu.async_copy(emb_sp.at[bhidx_v.at[r, h]],
                             irows_v.at[par, h], psem)

    def wait_row(par, psem):
        for h in range(2):
            pltpu.make_async_copy(emb_hbm.at[pl.ds(0, HALF)],
                                  urows_v.at[par, h], psem).wait()
            pltpu.make_async_copy(emb_hbm.at[pl.ds(0, HALF)],
                                  irows_v.at[par, h], psem).wait()

    def chunk_body(rc, carry):
        rowbase = base + rc * RC
        pltpu.sync_copy(fav_hbm.at[pl.ds(rowbase, RC)], fidx_v)
        pltpu.sync_copy(book_hbm.at[pl.ds(rowbase, RC)], bidx_v)
        for rr in range(RC):
            for h in range(2):
                for c in range(HALF // N_LANES):
                    sl = (rr, h, pl.ds(c * N_LANES, N_LANES))
                    fhidx_v[sl] = jnp.right_shift(fidx_v[sl], 1)
                    bhidx_v[sl] = jnp.right_shift(bidx_v[sl], 1)
        fire_row(0, 0, sem)

        def do_row(r, par):
            au, zu = _pool_row(r, fidx_v, fhidx_v, urows_v.at[par], s_v,
                               smu_v)
            ai, zi = _pool_row(r, bidx_v, bhidx_v, irows_v.at[par], s_v,
                               smi_v)
            u1, u2 = _bf16_halves(au)
            v1, v2 = _bf16_halves(ai)
            pu = u1 + u2
            pb = v1 + v2
            zzinv = _vrecip(jnp.full((N_LANES,), zu * zi, jnp.float32))
            dot = jnp.sum(pu * pb * zzinv)
            lane = lax.iota(jnp.int32, N_LANES)
            plsc.store_scatter(out_v,
                               [jnp.full((N_LANES,), rc * RC + r, jnp.int32)],
                               jnp.full((N_LANES,), dot, jnp.float32),
                               mask=lane == 0)

        def pair_body(q, carry2):
            r = q * 2
            fire_row(r + 1, 1, sem2)
            wait_row(0, sem)
            do_row(r, 0)

            @pl.when(q < RC // 2 - 1)
            def _prefetch():
                fire_row(r + 2, 0, sem)

            wait_row(1, sem2)
            do_row(r + 1, 1)
            return carry2

        lax.fori_loop(0, RC // 2, pair_body, 0)
        return carry

    lax.fori_loop(0, N_CHUNKS, chunk_body, 0)

    gb = gb_v[...]
    for k in range(ROWS_PER_W // N_LANES):
        cc, off = (k * N_LANES) // 128, (k * N_LANES) % 128
        o = (out_v[pl.ds(k * N_LANES, N_LANES)]
             + ub_v[cc, pl.ds(off, N_LANES)]
             + ib_v[cc, pl.ds(off, N_LANES)] + gb)
        out_v[pl.ds(k * N_LANES, N_LANES)] = o
    pltpu.sync_copy(out_v, out_hbm.at[pl.ds(base, ROWS_PER_W)])


@functools.partial(jax.jit, static_argnames=())
def kernel(user_idx, item_idx, fav_subjects, book_subjects, subj_emb,
           attn_w, attn_b, user_bias, item_bias, global_bias):
    del attn_b
    s1d = _score_table(subj_emb, attn_w.reshape(1, D)).reshape(N_SUBJECTS)
    sb = lax.bitcast_convert_type(s1d, jnp.uint32)
    s2 = lax.bitcast_convert_type(
        jnp.right_shift(sb[0::2], 16) | (sb[1::2] & jnp.uint32(0xFFFF0000)),
        jnp.int32)

    pad = jnp.zeros((B, LP - L), jnp.int32)
    favr = jnp.concatenate([fav_subjects, pad], axis=1).reshape(B, 2, HALF)
    bookr = jnp.concatenate([book_subjects, pad], axis=1).reshape(B, 2, HALF)
    uidx2 = user_idx.reshape(B // 128, 128)
    iidx2 = item_idx.reshape(B // 128, 128)
    ub_flat = user_bias.reshape(-1)
    ib_flat = item_bias.reshape(-1)
    gb16 = jnp.broadcast_to(global_bias.astype(jnp.float32), (N_LANES,))
    emb_il = (subj_emb.astype(jnp.bfloat16)
              .reshape(N_SUBJECTS // 2, 2, D)
              .transpose(0, 2, 1)
              .reshape(N_SUBJECTS // 2, 2 * D))

    mesh = plsc.VectorSubcoreMesh(core_axis_name="c", subcore_axis_name="s",
                                  num_cores=NC, num_subcores=NS)
    sc = pl.kernel(
        _sc_main,
        out_type=jax.ShapeDtypeStruct((B,), jnp.float32),
        mesh=mesh,
        compiler_params=pltpu.CompilerParams(needs_layout_passes=False,
                                             use_tc_tiling_on_sc=False,
                                             internal_scratch_in_bytes=64 * 1024),
        scratch_types=[
            pltpu.VMEM((N_SUBJECTS // 2,), jnp.int32),
            pltpu.VMEM((RC, 2, HALF), jnp.int32),
            pltpu.VMEM((RC, 2, HALF), jnp.int32),
            pltpu.VMEM((RC, 2, HALF), jnp.int32),
            pltpu.VMEM((RC, 2, HALF), jnp.int32),
            pltpu.VMEM((2, 2, HALF, 2 * D), jnp.bfloat16),
            pltpu.VMEM((2, 2, HALF, 2 * D), jnp.bfloat16),
            pltpu.VMEM((4, 128), jnp.int32),
            pltpu.VMEM((4, 128), jnp.int32),
            pltpu.VMEM((4, 128), jnp.float32),
            pltpu.VMEM((4, 128), jnp.float32),
            pltpu.VMEM((ROWS_PER_W,), jnp.float32),
            pltpu.VMEM((N_LANES,), jnp.float32),
            pltpu.VMEM((LP,), jnp.float32),
            pltpu.VMEM((LP,), jnp.float32),
            pltpu.VMEM_SHARED((N_SUBJECTS // 2, 2 * D), jnp.bfloat16),
            pltpu.SemaphoreType.DMA,
            pltpu.SemaphoreType.DMA,
        ],
    )
    return sc(s2, favr, bookr, uidx2, iidx2, emb_il,
              ub_flat, ib_flat, gb16)


# speedup vs baseline: 1.0173x; 1.0173x over previous
import functools

import jax
import jax.numpy as jnp
from jax import lax
from jax.experimental import pallas as pl
from jax.experimental.pallas import tpu as pltpu
from jax.experimental.pallas import tpu_sc as plsc

B = 16384
L = 200
LP = 224
HALF = 112
D = 16
N_SUBJECTS = 100000
N_LANES = 16
NC, NS = 2, 16
NW = NC * NS
ROWS_PER_W = B // NW
RC = 8
N_CHUNKS = ROWS_PER_W // RC
NEG = -1e30

_SBLK = 2000


def _score_table_body(e_ref, w_ref, o_ref):
    o_ref[...] = jnp.sum(e_ref[...] * w_ref[...], axis=1, keepdims=True)


def _score_table(subj_emb, attn_w_row):
    grid = N_SUBJECTS // _SBLK
    return pl.pallas_call(
        _score_table_body,
        grid=(grid,),
        in_specs=[
            pl.BlockSpec((_SBLK, D), lambda i: (i, 0)),
            pl.BlockSpec((1, D), lambda i: (0, 0)),
        ],
        out_specs=pl.BlockSpec((_SBLK, 1), lambda i: (i, 0)),
        out_shape=jax.ShapeDtypeStruct((N_SUBJECTS, 1), jnp.float32),
    )(subj_emb, attn_w_row)


_GATHER_DNUMS = lax.GatherDimensionNumbers(
    offset_dims=(), collapsed_slice_dims=(0,), start_index_map=(0,))


def _splat_lane(vec, j):
    idx = jnp.full((N_LANES, 1), j, jnp.int32)
    return lax.gather(vec, idx, _GATHER_DNUMS, (1,),
                      mode=lax.GatherScatterMode.PROMISE_IN_BOUNDS)


def _bf16_halves(a32):
    pr = plsc.bitcast(a32, jnp.int32)
    lo = plsc.bitcast(jnp.left_shift(pr, 16), jnp.float32)
    hi = plsc.bitcast(pr & jnp.int32(-65536), jnp.float32)
    return lo, hi


def _vrecip(x):
    xi = plsc.bitcast(x, jnp.int32)
    magic = jnp.full((N_LANES,), 0x7EF311C3, jnp.int32)
    r = plsc.bitcast(magic - xi, jnp.float32)
    for _ in range(3):
        r = r * (2.0 - x * r)
    return r


def _pool_row(r, idx_ref, idxh_ref, rows_ref, s_v):
    run_max = jnp.full((N_LANES,), NEG, jnp.float32)
    sms = []
    for h in range(2):
        for c in range(HALF // N_LANES):
            idx = idx_ref[r, h, pl.ds(c * N_LANES, N_LANES)]
            idxh = idxh_ref[r, h, pl.ds(c * N_LANES, N_LANES)]
            praw = plsc.load_gather(s_v, [idxh])
            sc = jnp.where(
                (idx & 1) == 0,
                plsc.bitcast(jnp.left_shift(praw, 16), jnp.float32),
                plsc.bitcast(praw & jnp.int32(-65536), jnp.float32))
            sm = jnp.where(idx != 0, sc, NEG)
            sms.append(sm)
            run_max = jnp.maximum(run_max, sm)
    m = jnp.max(run_max)
    zacc = jnp.zeros((N_LANES,), jnp.float32)
    accs = [jnp.zeros((2 * N_LANES,), jnp.bfloat16) for _ in range(4)]
    for k in range(LP // N_LANES):
        h, c = k // 7, k % 7
        idx = idx_ref[r, h, pl.ds(c * N_LANES, N_LANES)]
        sm = sms[k]
        e = jnp.exp(sm - m)
        e = jnp.where(sm != NEG, e, 0.0)
        zacc = zacc + e
        pv = idx & 1
        a = jnp.where(pv == 0, e, 0.0)
        b = e - a
        pp = (lax.shift_right_logical(plsc.bitcast(a, jnp.int32), 16)
              | (plsc.bitcast(b, jnp.int32) & jnp.int32(-65536)))
        for j in range(N_LANES):
            wb = plsc.bitcast(_splat_lane(pp, j), jnp.bfloat16)
            accs[j % 4] = (accs[j % 4]
                           + wb * rows_ref[h, c * N_LANES + j, :])
    z = jnp.maximum(jnp.sum(zacc), 1.0)
    acc = (accs[0] + accs[1]) + (accs[2] + accs[3])
    return acc, z


def _sc_main(s_hbm, fav_hbm, book_hbm,
             uidx_hbm, iidx_hbm, emb_hbm,
             ubias_hbm, ibias_hbm, gb_hbm, out_hbm,
             s_v, fidx_v, bidx_v, fhidx_v, bhidx_v, urows_v, irows_v,
             uidx_v, iidx_v, ub_v, ib_v, out_v, gb_v,
             emb_sp, sem, sem2):
    wid = lax.axis_index("s") * NC + lax.axis_index("c")
    base = wid * ROWS_PER_W

    pltpu.sync_copy(s_hbm, s_v)
    pltpu.sync_copy(gb_hbm, gb_v)
    pltpu.sync_copy(uidx_hbm.at[pl.ds(wid * 4, 4)], uidx_v)
    pltpu.sync_copy(iidx_hbm.at[pl.ds(wid * 4, 4)], iidx_v)
    descs = []
    for c in range(4):
        descs.append(pltpu.async_copy(ubias_hbm.at[uidx_v.at[c]],
                                      ub_v.at[c], sem))
        descs.append(pltpu.async_copy(ibias_hbm.at[iidx_v.at[c]],
                                      ib_v.at[c], sem))
    for d in descs:
        d.wait()

    @pl.when(lax.axis_index("s") == 0)
    def _stage_table():
        pltpu.sync_copy(emb_hbm, emb_sp)

    plsc.subcore_barrier()

    def fire_row(r, par, psem):
        for h in range(2):
            pltpu.async_copy(emb_sp.at[fhidx_v.at[r, h]],
                             urows_v.at[par, h], psem)
            pltpu.async_copy(emb_sp.at[bhidx_v.at[r, h]],
                             irows_v.at[par, h], psem)

    def wait_row(par, psem):
        for h in range(2):
            pltpu.make_async_copy(emb_hbm.at[pl.ds(0, HALF)],
                                  urows_v.at[par, h], psem).wait()
            pltpu.make_async_copy(emb_hbm.at[pl.ds(0, HALF)],
                                  irows_v.at[par, h], psem).wait()

    def chunk_body(rc, carry):
        rowbase = base + rc * RC
        pltpu.sync_copy(fav_hbm.at[pl.ds(rowbase, RC)], fidx_v)
        pltpu.sync_copy(book_hbm.at[pl.ds(rowbase, RC)], bidx_v)
        for rr in range(RC):
            for h in range(2):
                for c in range(HALF // N_LANES):
                    sl = (rr, h, pl.ds(c * N_LANES, N_LANES))
                    fhidx_v[sl] = jnp.right_shift(fidx_v[sl], 1)
                    bhidx_v[sl] = jnp.right_shift(bidx_v[sl], 1)
        fire_row(0, 0, sem)

        def do_row(r, par):
            au, zu = _pool_row(r, fidx_v, fhidx_v, urows_v.at[par], s_v)
            ai, zi = _pool_row(r, bidx_v, bhidx_v, irows_v.at[par], s_v)
            u1, u2 = _bf16_halves(au)
            v1, v2 = _bf16_halves(ai)
            pu = u1 + u2
            pb = v1 + v2
            zzinv = _vrecip(jnp.full((N_LANES,), zu * zi, jnp.float32))
            dot = jnp.sum(pu * pb * zzinv)
            lane = lax.iota(jnp.int32, N_LANES)
            plsc.store_scatter(out_v,
                               [jnp.full((N_LANES,), rc * RC + r, jnp.int32)],
                               jnp.full((N_LANES,), dot, jnp.float32),
                               mask=lane == 0)

        def pair_body(q, carry2):
            r = q * 2
            fire_row(r + 1, 1, sem2)
            wait_row(0, sem)
            do_row(r, 0)

            @pl.when(q < RC // 2 - 1)
            def _prefetch():
                fire_row(r + 2, 0, sem)

            wait_row(1, sem2)
            do_row(r + 1, 1)
            return carry2

        lax.fori_loop(0, RC // 2, pair_body, 0)
        return carry

    lax.fori_loop(0, N_CHUNKS, chunk_body, 0)

    gb = gb_v[...]
    for k in range(ROWS_PER_W // N_LANES):
        cc, off = (k * N_LANES) // 128, (k * N_LANES) % 128
        o = (out_v[pl.ds(k * N_LANES, N_LANES)]
             + ub_v[cc, pl.ds(off, N_LANES)]
             + ib_v[cc, pl.ds(off, N_LANES)] + gb)
        out_v[pl.ds(k * N_LANES, N_LANES)] = o
    pltpu.sync_copy(out_v, out_hbm.at[pl.ds(base, ROWS_PER_W)])


@functools.partial(jax.jit, static_argnames=())
def kernel(user_idx, item_idx, fav_subjects, book_subjects, subj_emb,
           attn_w, attn_b, user_bias, item_bias, global_bias):
    del attn_b
    s1d = _score_table(subj_emb, attn_w.reshape(1, D)).reshape(N_SUBJECTS)
    sb = lax.bitcast_convert_type(s1d, jnp.uint32)
    s2 = lax.bitcast_convert_type(
        jnp.right_shift(sb[0::2], 16) | (sb[1::2] & jnp.uint32(0xFFFF0000)),
        jnp.int32)

    pad = jnp.zeros((B, LP - L), jnp.int32)
    favr = jnp.concatenate([fav_subjects, pad], axis=1).reshape(B, 2, HALF)
    bookr = jnp.concatenate([book_subjects, pad], axis=1).reshape(B, 2, HALF)
    uidx2 = user_idx.reshape(B // 128, 128)
    iidx2 = item_idx.reshape(B // 128, 128)
    ub_flat = user_bias.reshape(-1)
    ib_flat = item_bias.reshape(-1)
    gb16 = jnp.broadcast_to(global_bias.astype(jnp.float32), (N_LANES,))
    emb_il = (subj_emb.astype(jnp.bfloat16)
              .reshape(N_SUBJECTS // 2, 2, D)
              .transpose(0, 2, 1)
              .reshape(N_SUBJECTS // 2, 2 * D))

    mesh = plsc.VectorSubcoreMesh(core_axis_name="c", subcore_axis_name="s",
                                  num_cores=NC, num_subcores=NS)
    sc = pl.kernel(
        _sc_main,
        out_type=jax.ShapeDtypeStruct((B,), jnp.float32),
        mesh=mesh,
        compiler_params=pltpu.CompilerParams(needs_layout_passes=False,
                                             use_tc_tiling_on_sc=False,
                                             internal_scratch_in_bytes=64 * 1024),
        scratch_types=[
            pltpu.VMEM((N_SUBJECTS // 2,), jnp.int32),
            pltpu.VMEM((RC, 2, HALF), jnp.int32),
            pltpu.VMEM((RC, 2, HALF), jnp.int32),
            pltpu.VMEM((RC, 2, HALF), jnp.int32),
            pltpu.VMEM((RC, 2, HALF), jnp.int32),
            pltpu.VMEM((2, 2, HALF, 2 * D), jnp.bfloat16),
            pltpu.VMEM((2, 2, HALF, 2 * D), jnp.bfloat16),
            pltpu.VMEM((4, 128), jnp.int32),
            pltpu.VMEM((4, 128), jnp.int32),
            pltpu.VMEM((4, 128), jnp.float32),
            pltpu.VMEM((4, 128), jnp.float32),
            pltpu.VMEM((ROWS_PER_W,), jnp.float32),
            pltpu.VMEM((N_LANES,), jnp.float32),
            pltpu.VMEM_SHARED((N_SUBJECTS // 2, 2 * D), jnp.bfloat16),
            pltpu.SemaphoreType.DMA,
            pltpu.SemaphoreType.DMA,
        ],
    )
    return sc(s2, favr, bookr, uidx2, iidx2, emb_il,
              ub_flat, ib_flat, gb16)


# speedup vs baseline: 1.1778x; 1.1578x over previous
import functools

import jax
import jax.numpy as jnp
from jax import lax
from jax.experimental import pallas as pl
from jax.experimental.pallas import tpu as pltpu
from jax.experimental.pallas import tpu_sc as plsc

B = 16384
L = 200
LP = 224
HALF = 112
D = 16
N_SUBJECTS = 100000
N_LANES = 16
NC, NS = 2, 16
NW = NC * NS
ROWS_PER_W = B // NW
RC = 8
N_CHUNKS = ROWS_PER_W // RC
NEG = -1e30

_SBLK = 2000


def _score_table_body(e_ref, w_ref, o_ref):
    o_ref[...] = jnp.sum(e_ref[...] * w_ref[...], axis=1, keepdims=True)


def _score_table(subj_emb, attn_w_row):
    grid = N_SUBJECTS // _SBLK
    return pl.pallas_call(
        _score_table_body,
        grid=(grid,),
        in_specs=[
            pl.BlockSpec((_SBLK, D), lambda i: (i, 0)),
            pl.BlockSpec((1, D), lambda i: (0, 0)),
        ],
        out_specs=pl.BlockSpec((_SBLK, 1), lambda i: (i, 0)),
        out_shape=jax.ShapeDtypeStruct((N_SUBJECTS, 1), jnp.float32),
    )(subj_emb, attn_w_row)


_GATHER_DNUMS = lax.GatherDimensionNumbers(
    offset_dims=(), collapsed_slice_dims=(0,), start_index_map=(0,))


def _splat_lane(vec, j):
    idx = jnp.full((N_LANES, 1), j, jnp.int32)
    return lax.gather(vec, idx, _GATHER_DNUMS, (1,),
                      mode=lax.GatherScatterMode.PROMISE_IN_BOUNDS)


def _bf16_halves(a32):
    pr = plsc.bitcast(a32, jnp.int32)
    lo = plsc.bitcast(jnp.left_shift(pr, 16), jnp.float32)
    hi = plsc.bitcast(pr & jnp.int32(-65536), jnp.float32)
    return lo, hi


def _vrecip(x):
    xi = plsc.bitcast(x, jnp.int32)
    magic = jnp.full((N_LANES,), 0x7EF311C3, jnp.int32)
    r = plsc.bitcast(magic - xi, jnp.float32)
    for _ in range(3):
        r = r * (2.0 - x * r)
    return r


def _pool_row(r, idx_ref, idxh_ref, rows_ref, s_v):
    run_max = jnp.full((N_LANES,), NEG, jnp.float32)
    sms = []
    for h, nch in ((0, 7), (1, 6)):
        for c in range(nch):
            idx = idx_ref[r, h, pl.ds(c * N_LANES, N_LANES)]
            idxh = idxh_ref[r, h, pl.ds(c * N_LANES, N_LANES)]
            praw = plsc.load_gather(s_v, [idxh])
            sc = jnp.where(
                (idx & 1) == 0,
                plsc.bitcast(jnp.left_shift(praw, 16), jnp.float32),
                plsc.bitcast(praw & jnp.int32(-65536), jnp.float32))
            sm = jnp.where(idx != 0, sc, NEG)
            sms.append(sm)
            run_max = jnp.maximum(run_max, sm)
    m = jnp.max(run_max)
    zacc = jnp.zeros((N_LANES,), jnp.float32)
    accs = [jnp.zeros((2 * N_LANES,), jnp.bfloat16) for _ in range(4)]
    for k in range(13):
        h, c = k // 7, k % 7
        idx = idx_ref[r, h, pl.ds(c * N_LANES, N_LANES)]
        sm = sms[k]
        e = jnp.exp(sm - m)
        e = jnp.where(sm != NEG, e, 0.0)
        zacc = zacc + e
        pv = idx & 1
        a = jnp.where(pv == 0, e, 0.0)
        b = e - a
        pp = (lax.shift_right_logical(plsc.bitcast(a, jnp.int32), 16)
              | (plsc.bitcast(b, jnp.int32) & jnp.int32(-65536)))
        for j in range(N_LANES):
            wb = plsc.bitcast(_splat_lane(pp, j), jnp.bfloat16)
            accs[j % 4] = (accs[j % 4]
                           + wb * rows_ref[h, c * N_LANES + j, :])
    z = jnp.maximum(jnp.sum(zacc), 1.0)
    acc = (accs[0] + accs[1]) + (accs[2] + accs[3])
    return acc, z


def _sc_main(s_hbm, fav_hbm, book_hbm,
             uidx_hbm, iidx_hbm, emb_hbm,
             ubias_hbm, ibias_hbm, gb_hbm, out_hbm,
             s_v, fidx_v, bidx_v, fhidx_v, bhidx_v, urows_v, irows_v,
             uidx_v, iidx_v, ub_v, ib_v, out_v, gb_v,
             emb_sp, sem, sem2):
    wid = lax.axis_index("s") * NC + lax.axis_index("c")
    base = wid * ROWS_PER_W

    pltpu.sync_copy(s_hbm, s_v)
    pltpu.sync_copy(gb_hbm, gb_v)
    pltpu.sync_copy(uidx_hbm.at[pl.ds(wid * 4, 4)], uidx_v)
    pltpu.sync_copy(iidx_hbm.at[pl.ds(wid * 4, 4)], iidx_v)
    descs = []
    for c in range(4):
        descs.append(pltpu.async_copy(ubias_hbm.at[uidx_v.at[c]],
                                      ub_v.at[c], sem))
        descs.append(pltpu.async_copy(ibias_hbm.at[iidx_v.at[c]],
                                      ib_v.at[c], sem))
    for d in descs:
        d.wait()

    @pl.when(lax.axis_index("s") == 0)
    def _stage_table():
        pltpu.sync_copy(emb_hbm, emb_sp)

    plsc.subcore_barrier()

    def fire_row(r, par, psem):
        for h, n in ((0, HALF), (1, 96)):
            pltpu.async_copy(emb_sp.at[fhidx_v.at[r, h, pl.ds(0, n)]],
                             urows_v.at[par, h, pl.ds(0, n)], psem)
            pltpu.async_copy(emb_sp.at[bhidx_v.at[r, h, pl.ds(0, n)]],
                             irows_v.at[par, h, pl.ds(0, n)], psem)

    def wait_row(par, psem):
        for h, n in ((0, HALF), (1, 96)):
            pltpu.make_async_copy(emb_hbm.at[pl.ds(0, n)],
                                  urows_v.at[par, h, pl.ds(0, n)],
                                  psem).wait()
            pltpu.make_async_copy(emb_hbm.at[pl.ds(0, n)],
                                  irows_v.at[par, h, pl.ds(0, n)],
                                  psem).wait()

    def chunk_body(rc, carry):
        rowbase = base + rc * RC
        pltpu.sync_copy(fav_hbm.at[pl.ds(rowbase, RC)], fidx_v)
        pltpu.sync_copy(book_hbm.at[pl.ds(rowbase, RC)], bidx_v)
        for rr in range(RC):
            for h, nch in ((0, 7), (1, 6)):
                for c in range(nch):
                    sl = (rr, h, pl.ds(c * N_LANES, N_LANES))
                    fhidx_v[sl] = jnp.right_shift(fidx_v[sl], 1)
                    bhidx_v[sl] = jnp.right_shift(bidx_v[sl], 1)
        fire_row(0, 0, sem)

        def do_row(r, par):
            au, zu = _pool_row(r, fidx_v, fhidx_v, urows_v.at[par], s_v)
            ai, zi = _pool_row(r, bidx_v, bhidx_v, irows_v.at[par], s_v)
            u1, u2 = _bf16_halves(au)
            v1, v2 = _bf16_halves(ai)
            pu = u1 + u2
            pb = v1 + v2
            zzinv = _vrecip(jnp.full((N_LANES,), zu * zi, jnp.float32))
            dot = jnp.sum(pu * pb * zzinv)
            lane = lax.iota(jnp.int32, N_LANES)
            plsc.store_scatter(out_v,
                               [jnp.full((N_LANES,), rc * RC + r, jnp.int32)],
                               jnp.full((N_LANES,), dot, jnp.float32),
                               mask=lane == 0)

        def pair_body(q, carry2):
            r = q * 2
            fire_row(r + 1, 1, sem2)
            wait_row(0, sem)
            do_row(r, 0)

            @pl.when(q < RC // 2 - 1)
            def _prefetch():
                fire_row(r + 2, 0, sem)

            wait_row(1, sem2)
            do_row(r + 1, 1)
            return carry2

        lax.fori_loop(0, RC // 2, pair_body, 0)
        return carry

    lax.fori_loop(0, N_CHUNKS, chunk_body, 0)

    gb = gb_v[...]
    for k in range(ROWS_PER_W // N_LANES):
        cc, off = (k * N_LANES) // 128, (k * N_LANES) % 128
        o = (out_v[pl.ds(k * N_LANES, N_LANES)]
             + ub_v[cc, pl.ds(off, N_LANES)]
             + ib_v[cc, pl.ds(off, N_LANES)] + gb)
        out_v[pl.ds(k * N_LANES, N_LANES)] = o
    pltpu.sync_copy(out_v, out_hbm.at[pl.ds(base, ROWS_PER_W)])


@functools.partial(jax.jit, static_argnames=())
def kernel(user_idx, item_idx, fav_subjects, book_subjects, subj_emb,
           attn_w, attn_b, user_bias, item_bias, global_bias):
    del attn_b
    s1d = _score_table(subj_emb, attn_w.reshape(1, D)).reshape(N_SUBJECTS)
    sb = lax.bitcast_convert_type(s1d, jnp.uint32)
    s2 = lax.bitcast_convert_type(
        jnp.right_shift(sb[0::2], 16) | (sb[1::2] & jnp.uint32(0xFFFF0000)),
        jnp.int32)

    pad = jnp.zeros((B, LP - L), jnp.int32)
    favr = jnp.concatenate([fav_subjects, pad], axis=1).reshape(B, 2, HALF)
    bookr = jnp.concatenate([book_subjects, pad], axis=1).reshape(B, 2, HALF)
    uidx2 = user_idx.reshape(B // 128, 128)
    iidx2 = item_idx.reshape(B // 128, 128)
    ub_flat = user_bias.reshape(-1)
    ib_flat = item_bias.reshape(-1)
    gb16 = jnp.broadcast_to(global_bias.astype(jnp.float32), (N_LANES,))
    emb_il = (subj_emb.astype(jnp.bfloat16)
              .reshape(N_SUBJECTS // 2, 2, D)
              .transpose(0, 2, 1)
              .reshape(N_SUBJECTS // 2, 2 * D))

    mesh = plsc.VectorSubcoreMesh(core_axis_name="c", subcore_axis_name="s",
                                  num_cores=NC, num_subcores=NS)
    sc = pl.kernel(
        _sc_main,
        out_type=jax.ShapeDtypeStruct((B,), jnp.float32),
        mesh=mesh,
        compiler_params=pltpu.CompilerParams(needs_layout_passes=False,
                                             use_tc_tiling_on_sc=False,
                                             internal_scratch_in_bytes=64 * 1024),
        scratch_types=[
            pltpu.VMEM((N_SUBJECTS // 2,), jnp.int32),
            pltpu.VMEM((RC, 2, HALF), jnp.int32),
            pltpu.VMEM((RC, 2, HALF), jnp.int32),
            pltpu.VMEM((RC, 2, HALF), jnp.int32),
            pltpu.VMEM((RC, 2, HALF), jnp.int32),
            pltpu.VMEM((2, 2, HALF, 2 * D), jnp.bfloat16),
            pltpu.VMEM((2, 2, HALF, 2 * D), jnp.bfloat16),
            pltpu.VMEM((4, 128), jnp.int32),
            pltpu.VMEM((4, 128), jnp.int32),
            pltpu.VMEM((4, 128), jnp.float32),
            pltpu.VMEM((4, 128), jnp.float32),
            pltpu.VMEM((ROWS_PER_W,), jnp.float32),
            pltpu.VMEM((N_LANES,), jnp.float32),
            pltpu.VMEM_SHARED((N_SUBJECTS // 2, 2 * D), jnp.bfloat16),
            pltpu.SemaphoreType.DMA,
            pltpu.SemaphoreType.DMA,
        ],
    )
    return sc(s2, favr, bookr, uidx2, iidx2, emb_il,
              ub_flat, ib_flat, gb16)


# speedup vs baseline: 1.1906x; 1.0108x over previous
import functools

import jax
import jax.numpy as jnp
from jax import lax
from jax.experimental import pallas as pl
from jax.experimental.pallas import tpu as pltpu
from jax.experimental.pallas import tpu_sc as plsc

B = 16384
L = 200
LP = 224
HALF = 112
D = 16
N_SUBJECTS = 100000
N_LANES = 16
NC, NS = 2, 16
NW = NC * NS
ROWS_PER_W = B // NW
RC = 8
N_CHUNKS = ROWS_PER_W // RC
NEG = -1e30

_SBLK = 2000


def _score_table_body(e_ref, w_ref, o_ref):
    o_ref[...] = jnp.sum(e_ref[...] * w_ref[...], axis=1, keepdims=True)


def _score_table(subj_emb, attn_w_row):
    grid = N_SUBJECTS // _SBLK
    return pl.pallas_call(
        _score_table_body,
        grid=(grid,),
        in_specs=[
            pl.BlockSpec((_SBLK, D), lambda i: (i, 0)),
            pl.BlockSpec((1, D), lambda i: (0, 0)),
        ],
        out_specs=pl.BlockSpec((_SBLK, 1), lambda i: (i, 0)),
        out_shape=jax.ShapeDtypeStruct((N_SUBJECTS, 1), jnp.float32),
    )(subj_emb, attn_w_row)


_GATHER_DNUMS = lax.GatherDimensionNumbers(
    offset_dims=(), collapsed_slice_dims=(0,), start_index_map=(0,))


def _splat_lane(vec, j):
    idx = jnp.full((N_LANES, 1), j, jnp.int32)
    return lax.gather(vec, idx, _GATHER_DNUMS, (1,),
                      mode=lax.GatherScatterMode.PROMISE_IN_BOUNDS)


def _bf16_halves(a32):
    pr = plsc.bitcast(a32, jnp.int32)
    lo = plsc.bitcast(jnp.left_shift(pr, 16), jnp.float32)
    hi = plsc.bitcast(pr & jnp.int32(-65536), jnp.float32)
    return lo, hi


def _vrecip(x):
    xi = plsc.bitcast(x, jnp.int32)
    magic = jnp.full((N_LANES,), 0x7EF311C3, jnp.int32)
    r = plsc.bitcast(magic - xi, jnp.float32)
    for _ in range(3):
        r = r * (2.0 - x * r)
    return r


def _pool_row(r, idx_ref, idxh_ref, rows_ref, s_v):
    run_max = jnp.full((N_LANES,), NEG, jnp.float32)
    sms = []
    for h, nch in ((0, 7), (1, 6)):
        for c in range(nch):
            idx = idx_ref[r, h, pl.ds(c * N_LANES, N_LANES)]
            idxh = idxh_ref[r, h, pl.ds(c * N_LANES, N_LANES)]
            praw = plsc.load_gather(s_v, [idxh])
            sc = jnp.where(
                (idx & 1) == 0,
                plsc.bitcast(jnp.left_shift(praw, 16), jnp.float32),
                plsc.bitcast(praw & jnp.int32(-65536), jnp.float32))
            sm = jnp.where(idx != 0, sc, NEG)
            sms.append(sm)
            run_max = jnp.maximum(run_max, sm)
    m = jnp.max(run_max)
    zacc = jnp.zeros((N_LANES,), jnp.float32)
    accs = [jnp.zeros((2 * N_LANES,), jnp.bfloat16) for _ in range(4)]
    for k in range(13):
        h, c = k // 7, k % 7
        idx = idx_ref[r, h, pl.ds(c * N_LANES, N_LANES)]
        sm = sms[k]
        e = jnp.exp(sm - m)
        e = jnp.where(sm != NEG, e, 0.0)
        zacc = zacc + e
        pv = idx & 1
        a = jnp.where(pv == 0, e, 0.0)
        b = e - a
        pp = (lax.shift_right_logical(plsc.bitcast(a, jnp.int32), 16)
              | (plsc.bitcast(b, jnp.int32) & jnp.int32(-65536)))
        for j in range(N_LANES):
            wb = plsc.bitcast(_splat_lane(pp, j), jnp.bfloat16)
            accs[j % 4] = (accs[j % 4]
                           + wb * rows_ref[h, c * N_LANES + j, :])
    z = jnp.maximum(jnp.sum(zacc), 1.0)
    acc = (accs[0] + accs[1]) + (accs[2] + accs[3])
    return acc, z


def _sc_main(s_hbm, fav_hbm, book_hbm,
             uidx_hbm, iidx_hbm, emb_hbm,
             ubias_hbm, ibias_hbm, gb_hbm, out_hbm,
             s_v, fidx_v, bidx_v, fhidx_v, bhidx_v, urows_v, irows_v,
             uidx_v, iidx_v, ub_v, ib_v, out_v, gb_v,
             emb_sp, sem, sem2):
    wid = lax.axis_index("s") * NC + lax.axis_index("c")
    base = wid * ROWS_PER_W

    pltpu.sync_copy(s_hbm, s_v)
    pltpu.sync_copy(gb_hbm, gb_v)
    pltpu.sync_copy(uidx_hbm.at[pl.ds(wid * 4, 4)], uidx_v)
    pltpu.sync_copy(iidx_hbm.at[pl.ds(wid * 4, 4)], iidx_v)
    descs = []
    for c in range(4):
        descs.append(pltpu.async_copy(ubias_hbm.at[uidx_v.at[c]],
                                      ub_v.at[c], sem))
        descs.append(pltpu.async_copy(ibias_hbm.at[iidx_v.at[c]],
                                      ib_v.at[c], sem))
    for d in descs:
        d.wait()

    @pl.when(lax.axis_index("s") == 0)
    def _stage_table():
        pltpu.sync_copy(emb_hbm, emb_sp)

    plsc.subcore_barrier()

    zrow = jnp.zeros((2 * D,), jnp.bfloat16)
    for par in range(2):
        for i in range(8):
            urows_v[par, 1, 88 + i, :] = zrow
            irows_v[par, 1, 88 + i, :] = zrow

    def fire_row(r, par, psem):
        for h, n in ((0, HALF), (1, 88)):
            pltpu.async_copy(emb_sp.at[fhidx_v.at[r, h, pl.ds(0, n)]],
                             urows_v.at[par, h, pl.ds(0, n)], psem)
            pltpu.async_copy(emb_sp.at[bhidx_v.at[r, h, pl.ds(0, n)]],
                             irows_v.at[par, h, pl.ds(0, n)], psem)

    def wait_row(par, psem):
        for h, n in ((0, HALF), (1, 88)):
            pltpu.make_async_copy(emb_hbm.at[pl.ds(0, n)],
                                  urows_v.at[par, h, pl.ds(0, n)],
                                  psem).wait()
            pltpu.make_async_copy(emb_hbm.at[pl.ds(0, n)],
                                  irows_v.at[par, h, pl.ds(0, n)],
                                  psem).wait()

    def chunk_body(rc, carry):
        rowbase = base + rc * RC
        pltpu.sync_copy(fav_hbm.at[pl.ds(rowbase, RC)], fidx_v)
        pltpu.sync_copy(book_hbm.at[pl.ds(rowbase, RC)], bidx_v)
        for rr in range(RC):
            for h, nch in ((0, 7), (1, 6)):
                for c in range(nch):
                    sl = (rr, h, pl.ds(c * N_LANES, N_LANES))
                    fhidx_v[sl] = jnp.right_shift(fidx_v[sl], 1)
                    bhidx_v[sl] = jnp.right_shift(bidx_v[sl], 1)
        fire_row(0, 0, sem)

        def do_row(r, par):
            au, zu = _pool_row(r, fidx_v, fhidx_v, urows_v.at[par], s_v)
            ai, zi = _pool_row(r, bidx_v, bhidx_v, irows_v.at[par], s_v)
            u1, u2 = _bf16_halves(au)
            v1, v2 = _bf16_halves(ai)
            pu = u1 + u2
            pb = v1 + v2
            zzinv = _vrecip(jnp.full((N_LANES,), zu * zi, jnp.float32))
            dot = jnp.sum(pu * pb * zzinv)
            lane = lax.iota(jnp.int32, N_LANES)
            plsc.store_scatter(out_v,
                               [jnp.full((N_LANES,), rc * RC + r, jnp.int32)],
                               jnp.full((N_LANES,), dot, jnp.float32),
                               mask=lane == 0)

        def pair_body(q, carry2):
            r = q * 2
            fire_row(r + 1, 1, sem2)
            wait_row(0, sem)
            do_row(r, 0)

            @pl.when(q < RC // 2 - 1)
            def _prefetch():
                fire_row(r + 2, 0, sem)

            wait_row(1, sem2)
            do_row(r + 1, 1)
            return carry2

        lax.fori_loop(0, RC // 2, pair_body, 0)
        return carry

    lax.fori_loop(0, N_CHUNKS, chunk_body, 0)

    gb = gb_v[...]
    for k in range(ROWS_PER_W // N_LANES):
        cc, off = (k * N_LANES) // 128, (k * N_LANES) % 128
        o = (out_v[pl.ds(k * N_LANES, N_LANES)]
             + ub_v[cc, pl.ds(off, N_LANES)]
             + ib_v[cc, pl.ds(off, N_LANES)] + gb)
        out_v[pl.ds(k * N_LANES, N_LANES)] = o
    pltpu.sync_copy(out_v, out_hbm.at[pl.ds(base, ROWS_PER_W)])


@functools.partial(jax.jit, static_argnames=())
def kernel(user_idx, item_idx, fav_subjects, book_subjects, subj_emb,
           attn_w, attn_b, user_bias, item_bias, global_bias):
    del attn_b
    s1d = _score_table(subj_emb, attn_w.reshape(1, D)).reshape(N_SUBJECTS)
    sb = lax.bitcast_convert_type(s1d, jnp.uint32)
    s2 = lax.bitcast_convert_type(
        jnp.right_shift(sb[0::2], 16) | (sb[1::2] & jnp.uint32(0xFFFF0000)),
        jnp.int32)

    pad = jnp.zeros((B, LP - L), jnp.int32)
    favr = jnp.concatenate([fav_subjects, pad], axis=1).reshape(B, 2, HALF)
    bookr = jnp.concatenate([book_subjects, pad], axis=1).reshape(B, 2, HALF)
    uidx2 = user_idx.reshape(B // 128, 128)
    iidx2 = item_idx.reshape(B // 128, 128)
    ub_flat = user_bias.reshape(-1)
    ib_flat = item_bias.reshape(-1)
    gb16 = jnp.broadcast_to(global_bias.astype(jnp.float32), (N_LANES,))
    emb_il = (subj_emb.astype(jnp.bfloat16)
              .reshape(N_SUBJECTS // 2, 2, D)
              .transpose(0, 2, 1)
              .reshape(N_SUBJECTS // 2, 2 * D))

    mesh = plsc.VectorSubcoreMesh(core_axis_name="c", subcore_axis_name="s",
                                  num_cores=NC, num_subcores=NS)
    sc = pl.kernel(
        _sc_main,
        out_type=jax.ShapeDtypeStruct((B,), jnp.float32),
        mesh=mesh,
        compiler_params=pltpu.CompilerParams(needs_layout_passes=False,
                                             use_tc_tiling_on_sc=False,
                                             internal_scratch_in_bytes=64 * 1024),
        scratch_types=[
            pltpu.VMEM((N_SUBJECTS // 2,), jnp.int32),
            pltpu.VMEM((RC, 2, HALF), jnp.int32),
            pltpu.VMEM((RC, 2, HALF), jnp.int32),
            pltpu.VMEM((RC, 2, HALF), jnp.int32),
            pltpu.VMEM((RC, 2, HALF), jnp.int32),
            pltpu.VMEM((2, 2, HALF, 2 * D), jnp.bfloat16),
            pltpu.VMEM((2, 2, HALF, 2 * D), jnp.bfloat16),
            pltpu.VMEM((4, 128), jnp.int32),
            pltpu.VMEM((4, 128), jnp.int32),
            pltpu.VMEM((4, 128), jnp.float32),
            pltpu.VMEM((4, 128), jnp.float32),
            pltpu.VMEM((ROWS_PER_W,), jnp.float32),
            pltpu.VMEM((N_LANES,), jnp.float32),
            pltpu.VMEM_SHARED((N_SUBJECTS // 2, 2 * D), jnp.bfloat16),
            pltpu.SemaphoreType.DMA,
            pltpu.SemaphoreType.DMA,
        ],
    )
    return sc(s2, favr, bookr, uidx2, iidx2, emb_il,
              ub_flat, ib_flat, gb16)
